# baseline (speedup 1.0000x reference)
import functools

import jax
import jax.numpy as jnp
from jax import lax
from jax.experimental import pallas as pl
from jax.experimental.pallas import tpu as pltpu
from jax.experimental.pallas import tpu_sc as plsc

_NE = 1024
_D = 128
_N = 32 * 1024
_BT = 512
_GRID = _N // _BT
_SCALE = 1.25 / (_N * _D)

_NW = 32
_RPW = _N // _NW
_CH = 128
_NCH = _RPW // _CH


def _rowsum128(a):
    s = a[:, 0:8]
    for i in range(1, 16):
        s = s + a[:, 8 * i:8 * (i + 1)]
    w = 8
    while w > 1:
        w //= 2
        s = s[:, :w] + s[:, w:]
    return s


def _tc_body(x_ref, cb_ref, cbn_ref, idx_ref, loss_ref):
    i = pl.program_id(0)
    xb = x_ref[...]
    cb = cb_ref[...]
    ab = lax.dot_general(xb, cb, (((1,), (1,)), ((), ())),
                         preferred_element_type=jnp.float32)
    rn = _rowsum128(xb * xb)
    cbn = cbn_ref[...]
    d2 = rn - 2.0 * ab + cbn
    dist = jnp.sqrt(jnp.maximum(d2, 0.0))
    minv = jnp.min(dist, axis=1, keepdims=True)
    ids = lax.broadcasted_iota(jnp.int32, dist.shape, 1)
    idx = jnp.min(jnp.where(dist == minv, ids, jnp.int32(_NE)), axis=1)
    idx_ref[0, 0, :] = idx

    d2min = jnp.min(jnp.maximum(d2, 0.0), axis=1)
    part = jnp.sum(d2min)

    @pl.when(i == 0)
    def _init():
        loss_ref[0, 0] = 0.0

    loss_ref[0, 0] += part

    @pl.when(i == _GRID - 1)
    def _fini():
        loss_ref[0, 0] = loss_ref[0, 0] * _SCALE


_tc_call = pl.pallas_call(
    _tc_body,
    grid=(_GRID,),
    in_specs=[
        pl.BlockSpec((_BT, _D), lambda i: (i, 0)),
        pl.BlockSpec((_NE, _D), lambda i: (0, 0)),
        pl.BlockSpec((1, _NE), lambda i: (0, 0)),
    ],
    out_specs=[
        pl.BlockSpec((1, 1, _BT), lambda i: (i, 0, 0)),
        pl.BlockSpec(memory_space=pltpu.SMEM),
    ],
    out_shape=[
        jax.ShapeDtypeStruct((_GRID, 1, _BT), jnp.int32),
        jax.ShapeDtypeStruct((1, 1), jnp.float32),
    ],
)


def _sc_gather(codebook, idx2d):
    mesh = plsc.VectorSubcoreMesh(core_axis_name="c", subcore_axis_name="s")

    @functools.partial(
        pl.kernel,
        mesh=mesh,
        out_type=jax.ShapeDtypeStruct((_N, _D), jnp.float32),
        scratch_types=[
            pltpu.VMEM((_NCH, _CH), jnp.int32),
            pltpu.VMEM((_CH, _D), jnp.float32),
            pltpu.VMEM((_CH, _D), jnp.float32),
            pltpu.SemaphoreType.DMA,
            pltpu.SemaphoreType.DMA,
        ],
    )
    def k(cb_hbm, idx_hbm, out_hbm, idx_v, r0, r1, s0, s1):
        wid = lax.axis_index("s") * 2 + lax.axis_index("c")
        pltpu.sync_copy(idx_hbm.at[pl.ds(wid * _NCH, _NCH)], idx_v)
        bufs = (r0, r1)
        sems = (s0, s1)
        copies = [pltpu.async_copy(cb_hbm.at[idx_v.at[0]], r0, s0)]
        for j in range(_NCH):
            if j + 1 < _NCH:
                copies.append(pltpu.async_copy(
                    cb_hbm.at[idx_v.at[j + 1]], bufs[(j + 1) % 2],
                    sems[(j + 1) % 2]))
            copies[j].wait()
            pltpu.sync_copy(bufs[j % 2],
                            out_hbm.at[pl.ds(wid * _RPW + j * _CH, _CH)])

    return k(codebook, idx2d)


def kernel(x, codebook):
    flat = x.reshape(_N, _D)
    cbn = jnp.sum(codebook * codebook, axis=1)[None, :]
    idx3, acc = _tc_call(flat, codebook, cbn)
    indices = idx3.reshape(_N)
    loss = acc[0, 0]
    q = _sc_gather(codebook, indices.reshape(_N // _D, _D))
    return q.reshape(x.shape), loss, indices


# speedup vs baseline: 1.0368x; 1.0368x over previous
import functools

import jax
import jax.numpy as jnp
from jax import lax
from jax.experimental import pallas as pl
from jax.experimental.pallas import tpu as pltpu
from jax.experimental.pallas import tpu_sc as plsc

_NE = 1024
_D = 128
_N = 32 * 1024
_BT = 512
_GRID = _N // _BT
_SCALE = 1.25 / (_N * _D)

_NW = 32
_RPW = _N // _NW
_CH = 128
_NCH = _RPW // _CH


def _rowsum128(a):
    s = a[:, 0:8]
    for i in range(1, 16):
        s = s + a[:, 8 * i:8 * (i + 1)]
    w = 8
    while w > 1:
        w //= 2
        s = s[:, :w] + s[:, w:]
    return s


def _tc_body(x_ref, cbm2_ref, cbn_ref, idx_ref, loss_ref):
    i = pl.program_id(0)
    xb = x_ref[...]
    cbm2 = cbm2_ref[...]
    ab2 = lax.dot_general(xb, cbm2, (((1,), (1,)), ((), ())),
                          preferred_element_type=jnp.float32)
    rn = _rowsum128(xb * xb)
    cbn = cbn_ref[...]
    d2 = (rn + ab2) + cbn
    dist = jnp.sqrt(jnp.maximum(d2, 0.0))
    minv = jnp.min(dist, axis=1, keepdims=True)
    ids = lax.broadcasted_iota(jnp.int32, dist.shape, 1)
    idx = jnp.min(jnp.where(dist == minv, ids, jnp.int32(_NE)), axis=1)
    idx_ref[0, 0, :] = idx

    mv = minv[:, 0]
    part = jnp.sum(mv * mv)

    @pl.when(i == 0)
    def _init():
        loss_ref[0, 0] = 0.0

    loss_ref[0, 0] += part

    @pl.when(i == _GRID - 1)
    def _fini():
        loss_ref[0, 0] = loss_ref[0, 0] * _SCALE


_tc_call = pl.pallas_call(
    _tc_body,
    grid=(_GRID,),
    in_specs=[
        pl.BlockSpec((_BT, _D), lambda i: (i, 0)),
        pl.BlockSpec((_NE, _D), lambda i: (0, 0)),
        pl.BlockSpec((1, _NE), lambda i: (0, 0)),
    ],
    out_specs=[
        pl.BlockSpec((1, 1, _BT), lambda i: (i, 0, 0)),
        pl.BlockSpec(memory_space=pltpu.SMEM),
    ],
    out_shape=[
        jax.ShapeDtypeStruct((_GRID, 1, _BT), jnp.int32),
        jax.ShapeDtypeStruct((1, 1), jnp.float32),
    ],
)


def _sc_gather(codebook, idx2d):
    mesh = plsc.VectorSubcoreMesh(core_axis_name="c", subcore_axis_name="s")

    @functools.partial(
        pl.kernel,
        mesh=mesh,
        out_type=jax.ShapeDtypeStruct((_N, _D), jnp.float32),
        scratch_types=[
            pltpu.VMEM((_NCH, _CH), jnp.int32),
            pltpu.VMEM((_CH, _D), jnp.float32),
            pltpu.VMEM((_CH, _D), jnp.float32),
            pltpu.SemaphoreType.DMA,
            pltpu.SemaphoreType.DMA,
        ],
    )
    def k(cb_hbm, idx_hbm, out_hbm, idx_v, r0, r1, s0, s1):
        wid = lax.axis_index("s") * 2 + lax.axis_index("c")
        pltpu.sync_copy(idx_hbm.at[pl.ds(wid * _NCH, _NCH)], idx_v)
        bufs = (r0, r1)
        sems = (s0, s1)
        copies = [pltpu.async_copy(cb_hbm.at[idx_v.at[0]], r0, s0)]
        for j in range(_NCH):
            if j + 1 < _NCH:
                copies.append(pltpu.async_copy(
                    cb_hbm.at[idx_v.at[j + 1]], bufs[(j + 1) % 2],
                    sems[(j + 1) % 2]))
            copies[j].wait()
            pltpu.sync_copy(bufs[j % 2],
                            out_hbm.at[pl.ds(wid * _RPW + j * _CH, _CH)])

    return k(codebook, idx2d)


def kernel(x, codebook):
    flat = x.reshape(_N, _D)
    cbn = jnp.sum(codebook * codebook, axis=1)[None, :]
    idx3, acc = _tc_call(flat, -2.0 * codebook, cbn)
    indices = idx3.reshape(_N)
    loss = acc[0, 0]
    q = _sc_gather(codebook, indices.reshape(_N // _D, _D))
    return q.reshape(x.shape), loss, indices


# speedup vs baseline: 1.7085x; 1.6479x over previous
import functools

import jax
import jax.numpy as jnp
from jax import lax
from jax.experimental import pallas as pl
from jax.experimental.pallas import tpu as pltpu
from jax.experimental.pallas import tpu_sc as plsc

_NE = 1024
_D = 128
_N = 32 * 1024
_BT = 512
_GRID = _N // _BT
_SCALE = 1.25 / (_N * _D)

_NW = 32
_RPW = _N // _NW
_CH = 128
_NCH = _RPW // _CH


def _tc_body(x_ref, cbm2_ref, cbn_ref, idx_ref, loss_ref):
    i = pl.program_id(0)
    xb = x_ref[...]
    cbm2 = cbm2_ref[...]
    ab2 = lax.dot_general(cbm2, xb, (((1,), (1,)), ((), ())),
                          preferred_element_type=jnp.float32)
    xsqT = (xb * xb).T
    s = xsqT[0:8, :]
    for j in range(1, 16):
        s = s + xsqT[8 * j:8 * (j + 1), :]
    w = 8
    while w > 1:
        w //= 2
        s = s[:w, :] + s[w:, :]
    rnT = s
    cbnT = cbn_ref[...]
    d2 = (rnT + ab2) + cbnT
    dist = jnp.sqrt(jnp.maximum(d2, 0.0))
    minv = jnp.min(dist, axis=0, keepdims=True)
    ids = lax.broadcasted_iota(jnp.int32, dist.shape, 0)
    idx = jnp.min(jnp.where(dist == minv, ids, jnp.int32(_NE)), axis=0)
    idx_ref[0, 0, :] = idx

    mv = minv[0, :]
    part = jnp.sum(mv * mv)

    @pl.when(i == 0)
    def _init():
        loss_ref[0, 0] = 0.0

    loss_ref[0, 0] += part

    @pl.when(i == _GRID - 1)
    def _fini():
        loss_ref[0, 0] = loss_ref[0, 0] * _SCALE


_tc_call = pl.pallas_call(
    _tc_body,
    grid=(_GRID,),
    in_specs=[
        pl.BlockSpec((_BT, _D), lambda i: (i, 0)),
        pl.BlockSpec((_NE, _D), lambda i: (0, 0)),
        pl.BlockSpec((_NE, 1), lambda i: (0, 0)),
    ],
    out_specs=[
        pl.BlockSpec((1, 1, _BT), lambda i: (i, 0, 0)),
        pl.BlockSpec(memory_space=pltpu.SMEM),
    ],
    out_shape=[
        jax.ShapeDtypeStruct((_GRID, 1, _BT), jnp.int32),
        jax.ShapeDtypeStruct((1, 1), jnp.float32),
    ],
)


def _sc_gather(codebook, idx2d):
    mesh = plsc.VectorSubcoreMesh(core_axis_name="c", subcore_axis_name="s")

    @functools.partial(
        pl.kernel,
        mesh=mesh,
        out_type=jax.ShapeDtypeStruct((_N, _D), jnp.float32),
        scratch_types=[
            pltpu.VMEM((_NCH, _CH), jnp.int32),
            pltpu.VMEM((_CH, _D), jnp.float32),
            pltpu.VMEM((_CH, _D), jnp.float32),
            pltpu.SemaphoreType.DMA,
            pltpu.SemaphoreType.DMA,
        ],
    )
    def k(cb_hbm, idx_hbm, out_hbm, idx_v, r0, r1, s0, s1):
        wid = lax.axis_index("s") * 2 + lax.axis_index("c")
        pltpu.sync_copy(idx_hbm.at[pl.ds(wid * _NCH, _NCH)], idx_v)
        bufs = (r0, r1)
        sems = (s0, s1)
        copies = [pltpu.async_copy(cb_hbm.at[idx_v.at[0]], r0, s0)]
        for j in range(_NCH):
            if j + 1 < _NCH:
                copies.append(pltpu.async_copy(
                    cb_hbm.at[idx_v.at[j + 1]], bufs[(j + 1) % 2],
                    sems[(j + 1) % 2]))
            copies[j].wait()
            pltpu.sync_copy(bufs[j % 2],
                            out_hbm.at[pl.ds(wid * _RPW + j * _CH, _CH)])

    return k(codebook, idx2d)


def kernel(x, codebook):
    flat = x.reshape(_N, _D)
    cbn = jnp.sum(codebook * codebook, axis=1)[:, None]
    idx3, acc = _tc_call(flat, -2.0 * codebook, cbn)
    indices = idx3.reshape(_N)
    loss = acc[0, 0]
    q = _sc_gather(codebook, indices.reshape(_N // _D, _D))
    return q.reshape(x.shape), loss, indices
